# baseline (speedup 1.0000x reference)
import functools

import jax
import jax.numpy as jnp
from jax import lax
from jax.experimental import pallas as pl
from jax.experimental.pallas import tpu as pltpu
from jax.experimental.pallas import tpu_sc as plsc

_NUM_EMB = 4200
_EMB_DIM = 128
_PAD = 1
_BSZ = 16
_SEQ = 4096
_HALF = _SEQ // 2
_STRIPE = _SEQ // 16
_ZCHUNK = 128
_ZREMROWS = _ZCHUNK // 2

_CSIZES = (2048, 1024, 512, 256, 128, 64, 32, 16, 8, 4, 2, 1)
_ZSIZES = (64, 32, 16, 8, 4, 2, 1)


def _body(lengths_hbm, weight_hbm, out_hbm, wslice, zshared, ztile, len_v,
          setup_sem, main_sem):
    cid = lax.axis_index("c")
    sid = lax.axis_index("s")
    lo = (jnp.bitwise_and(sid + cid, 1)) * _HALF

    pltpu.async_copy(
        weight_hbm.at[pl.ds(_PAD + 1 + sid * _STRIPE, _STRIPE), :],
        wslice.at[pl.ds(sid * _STRIPE, _STRIPE), :],
        setup_sem,
    )
    pltpu.async_copy(lengths_hbm, len_v, setup_sem)

    zeros16 = jnp.zeros((16,), jnp.float32)

    def _zero_row(r, carry):
        for kk in range(_EMB_DIM // 16):
            ztile[r, pl.ds(kk * 16, 16)] = zeros16
        return carry

    lax.fori_loop(0, _ZCHUNK, _zero_row, 0)

    @pl.when(sid == 0)
    def _publish_zeros():
        pltpu.sync_copy(ztile.at[pl.ds(0, _ZREMROWS), :], zshared)

    pltpu.make_async_copy(
        weight_hbm.at[pl.ds(0, _STRIPE), :],
        wslice.at[pl.ds(sid * _STRIPE, _STRIPE), :],
        setup_sem,
    ).wait()
    pltpu.make_async_copy(lengths_hbm, len_v, setup_sem).wait()
    plsc.subcore_barrier()

    b = sid
    bvec = jnp.broadcast_to(b, (16,)).astype(jnp.int32)
    length = plsc.load_gather(len_v, [bvec])[0]

    cnt = jnp.clip(length - lo, 0, _HALF)
    zcnt = _HALF - cnt
    zrem = jnp.bitwise_and(zcnt, _ZCHUNK - 1)
    nz = lax.shift_right_logical(zcnt, 7)

    off = lo
    for size in _CSIZES:
        take = jnp.bitwise_and(cnt, size)
        cur = off

        @pl.when(take > 0)
        def _copy(cur=cur, size=size):
            pltpu.async_copy(
                wslice.at[pl.ds(cur, size), :],
                out_hbm.at[b, pl.ds(cur, size), :],
                main_sem,
            )

        off = off + take

    for size in _ZSIZES:
        take = jnp.bitwise_and(zrem, size)
        cur = off

        @pl.when(take > 0)
        def _fill(cur=cur, size=size):
            pltpu.async_copy(
                zshared.at[pl.ds(0, size), :],
                out_hbm.at[b, pl.ds(cur, size), :],
                main_sem,
            )

        off = off + take

    zoff = off

    def _zchunk(i, carry):
        pltpu.async_copy(
            ztile,
            out_hbm.at[b, pl.ds(zoff + i * _ZCHUNK, _ZCHUNK), :],
            main_sem,
        )
        return carry

    lax.fori_loop(0, nz, _zchunk, 0)

    pltpu.make_async_copy(
        out_hbm.at[b, pl.ds(lo, _HALF), :],
        wslice.at[pl.ds(0, _HALF), :],
        main_sem,
    ).wait()


@jax.jit
def _positional_embedding(lengths, weight):
    mesh = plsc.VectorSubcoreMesh(
        core_axis_name="c", subcore_axis_name="s", num_cores=2, num_subcores=16
    )
    return pl.kernel(
        _body,
        out_type=jax.ShapeDtypeStruct((_BSZ, _SEQ, _EMB_DIM), jnp.float32),
        mesh=mesh,
        compiler_params=pltpu.CompilerParams(
            use_tc_tiling_on_sc=False, needs_layout_passes=False
        ),
        scratch_types=[
            pltpu.VMEM_SHARED((_SEQ, _EMB_DIM), jnp.float32),
            pltpu.VMEM_SHARED((_ZREMROWS, _EMB_DIM), jnp.float32),
            pltpu.VMEM((_ZCHUNK, _EMB_DIM), jnp.float32),
            pltpu.VMEM((16,), jnp.int32),
            pltpu.SemaphoreType.DMA,
            pltpu.SemaphoreType.DMA,
        ],
    )(lengths, weight)


def kernel(input, lengths, weight):
    del input
    return _positional_embedding(lengths, weight)


# speedup vs baseline: 1.0103x; 1.0103x over previous
import functools

import jax
import jax.numpy as jnp
from jax import lax
from jax.experimental import pallas as pl
from jax.experimental.pallas import tpu as pltpu
from jax.experimental.pallas import tpu_sc as plsc

_NUM_EMB = 4200
_EMB_DIM = 128
_PAD = 1
_BSZ = 16
_SEQ = 4096
_HALF = _SEQ // 2
_STRIPE = _SEQ // 16
_CHUNK = 128

_CSIZES = (2048, 1024, 512, 256, 128)


def _body(lengths_hbm, weight_hbm, out_hbm, wslice, ztile, bbuf, len_v,
          setup_sem, main_sem):
    cid = lax.axis_index("c")
    sid = lax.axis_index("s")
    lo = jnp.bitwise_and(sid + cid, 1) * _HALF

    pltpu.async_copy(
        weight_hbm.at[pl.ds(_PAD + 1 + sid * _STRIPE, _STRIPE), :],
        wslice.at[pl.ds(sid * _STRIPE, _STRIPE), :],
        setup_sem,
    )
    pltpu.async_copy(lengths_hbm, len_v, setup_sem)

    zeros16 = jnp.zeros((16,), jnp.float32)

    def _zero_row(r, carry):
        for kk in range(_EMB_DIM // 16):
            ztile[r, pl.ds(kk * 16, 16)] = zeros16
        return carry

    lax.fori_loop(0, _CHUNK, _zero_row, 0)

    pltpu.make_async_copy(
        weight_hbm.at[pl.ds(0, _STRIPE), :],
        wslice.at[pl.ds(sid * _STRIPE, _STRIPE), :],
        setup_sem,
    ).wait()
    pltpu.make_async_copy(lengths_hbm, len_v, setup_sem).wait()
    plsc.subcore_barrier()

    b = sid
    bvec = jnp.broadcast_to(b, (16,)).astype(jnp.int32)
    length = plsc.load_gather(len_v, [bvec])[0]

    cnt = jnp.clip(length - lo, 0, _HALF)
    rem = jnp.bitwise_and(cnt, _CHUNK - 1)
    aligned = cnt - rem

    off = lo
    for size in _CSIZES:
        take = jnp.bitwise_and(aligned, size)
        cur = off

        @pl.when(take > 0)
        def _copy(cur=cur, size=size):
            pltpu.async_copy(
                wslice.at[pl.ds(cur, size), :],
                out_hbm.at[b, pl.ds(cur, size), :],
                main_sem,
            )

        off = off + take

    @pl.when(rem > 0)
    def _boundary():
        pltpu.sync_copy(wslice.at[pl.ds(lo + aligned, _CHUNK), :], bbuf)

        def _zero_tail(r, carry):
            for kk in range(_EMB_DIM // 16):
                bbuf[r, pl.ds(kk * 16, 16)] = zeros16
            return carry

        lax.fori_loop(rem, _CHUNK, _zero_tail, 0)
        pltpu.async_copy(
            bbuf, out_hbm.at[b, pl.ds(lo + aligned, _CHUNK), :], main_sem
        )

    zoff = lo + aligned + jnp.where(rem > 0, _CHUNK, 0)
    nz = lax.shift_right_logical(lo + _HALF - zoff, 7)

    def _zchunk(i, carry):
        pltpu.async_copy(
            ztile,
            out_hbm.at[b, pl.ds(zoff + i * _CHUNK, _CHUNK), :],
            main_sem,
        )
        return carry

    lax.fori_loop(0, nz, _zchunk, 0)

    pltpu.make_async_copy(
        out_hbm.at[b, pl.ds(lo, _HALF), :],
        wslice.at[pl.ds(0, _HALF), :],
        main_sem,
    ).wait()


@jax.jit
def _positional_embedding(lengths, weight):
    mesh = plsc.VectorSubcoreMesh(
        core_axis_name="c", subcore_axis_name="s", num_cores=2, num_subcores=16
    )
    return pl.kernel(
        _body,
        out_type=jax.ShapeDtypeStruct((_BSZ, _SEQ, _EMB_DIM), jnp.float32),
        mesh=mesh,
        compiler_params=pltpu.CompilerParams(
            use_tc_tiling_on_sc=False, needs_layout_passes=False
        ),
        scratch_types=[
            pltpu.VMEM_SHARED((_SEQ, _EMB_DIM), jnp.float32),
            pltpu.VMEM((_CHUNK, _EMB_DIM), jnp.float32),
            pltpu.VMEM((_CHUNK, _EMB_DIM), jnp.float32),
            pltpu.VMEM((16,), jnp.int32),
            pltpu.SemaphoreType.DMA,
            pltpu.SemaphoreType.DMA,
        ],
    )(lengths, weight)


def kernel(input, lengths, weight):
    del input
    return _positional_embedding(lengths, weight)


# speedup vs baseline: 1.1825x; 1.1705x over previous
import functools

import jax
import jax.numpy as jnp
from jax import lax
from jax.experimental import pallas as pl
from jax.experimental.pallas import tpu as pltpu
from jax.experimental.pallas import tpu_sc as plsc

_NUM_EMB = 4200
_EMB_DIM = 128
_PAD = 1
_BSZ = 16
_SEQ = 4096
_CHUNK = 128
_NCHUNKS = _SEQ // _CHUNK

_BSIZES = (64, 32, 16, 8, 4, 2, 1)


def _body(lengths_hbm, weight_hbm, out_hbm, wchunk, ztile, len_v,
          setup_sem, main_sem):
    cid = lax.axis_index("c")
    sid = lax.axis_index("s")
    g0 = (sid * 2 + cid) * _CHUNK

    pltpu.async_copy(
        weight_hbm.at[pl.ds(_PAD + 1 + g0, _CHUNK), :], wchunk, setup_sem
    )
    pltpu.async_copy(lengths_hbm, len_v, setup_sem)

    zeros16 = jnp.zeros((16,), jnp.float32)

    def _zero_row(r, carry):
        for kk in range(_EMB_DIM // 16):
            ztile[r, pl.ds(kk * 16, 16)] = zeros16
        return carry

    lax.fori_loop(0, _CHUNK, _zero_row, 0)

    pltpu.make_async_copy(
        weight_hbm.at[pl.ds(0, _CHUNK), :], wchunk, setup_sem
    ).wait()
    pltpu.make_async_copy(lengths_hbm, len_v, setup_sem).wait()

    def _per_batch(b, carry):
        bvec = jnp.broadcast_to(b, (16,)).astype(jnp.int32)
        length = plsc.load_gather(len_v, [bvec])[0]
        cnt = jnp.clip(length - g0, 0, _CHUNK)

        @pl.when(cnt == _CHUNK)
        def _full():
            pltpu.async_copy(
                wchunk, out_hbm.at[b, pl.ds(g0, _CHUNK), :], main_sem
            )

        @pl.when(cnt == 0)
        def _zero():
            pltpu.async_copy(
                ztile, out_hbm.at[b, pl.ds(g0, _CHUNK), :], main_sem
            )

        @pl.when(jnp.logical_and(cnt > 0, cnt < _CHUNK))
        def _boundary():
            off = g0
            rest = _CHUNK - cnt
            for size in _BSIZES:
                take = jnp.bitwise_and(cnt, size)
                cur = off

                @pl.when(take > 0)
                def _copybits(cur=cur, size=size):
                    pltpu.async_copy(
                        wchunk.at[pl.ds(cur - g0, size), :],
                        out_hbm.at[b, pl.ds(cur, size), :],
                        main_sem,
                    )

                off = off + take
            for size in _BSIZES:
                take = jnp.bitwise_and(rest, size)
                cur = off

                @pl.when(take > 0)
                def _zerobits(cur=cur, size=size):
                    pltpu.async_copy(
                        ztile.at[pl.ds(0, size), :],
                        out_hbm.at[b, pl.ds(cur, size), :],
                        main_sem,
                    )

                off = off + take

        return carry

    lax.fori_loop(0, _BSZ, _per_batch, 0)

    pltpu.make_async_copy(
        out_hbm.at[pl.ds(0, _BSZ), pl.ds(0, _CHUNK), :],
        out_hbm.at[pl.ds(0, _BSZ), pl.ds(0, _CHUNK), :],
        main_sem,
    ).wait()


@jax.jit
def _positional_embedding(lengths, weight):
    mesh = plsc.VectorSubcoreMesh(
        core_axis_name="c", subcore_axis_name="s", num_cores=2, num_subcores=16
    )
    return pl.kernel(
        _body,
        out_type=jax.ShapeDtypeStruct((_BSZ, _SEQ, _EMB_DIM), jnp.float32),
        mesh=mesh,
        compiler_params=pltpu.CompilerParams(
            use_tc_tiling_on_sc=False, needs_layout_passes=False
        ),
        scratch_types=[
            pltpu.VMEM((_CHUNK, _EMB_DIM), jnp.float32),
            pltpu.VMEM((_CHUNK, _EMB_DIM), jnp.float32),
            pltpu.VMEM((16,), jnp.int32),
            pltpu.SemaphoreType.DMA,
            pltpu.SemaphoreType.DMA,
        ],
    )(lengths, weight)


def kernel(input, lengths, weight):
    del input
    return _positional_embedding(lengths, weight)


# speedup vs baseline: 1.1906x; 1.0068x over previous
import jax
import jax.numpy as jnp
from jax import lax
from jax.experimental import pallas as pl
from jax.experimental.pallas import tpu as pltpu
from jax.experimental.pallas import tpu_sc as plsc

_NUM_EMB = 4200
_EMB_DIM = 128
_PAD = 1
_BSZ = 16
_SEQ = 4096
_CHUNK = 128
_NCHUNKS = _SEQ // _CHUNK

_BSIZES = (64, 32, 16, 8, 4, 2, 1)


def _body(lengths_hbm, weight_hbm, out_hbm, wchunk, ztile, len_v,
          setup_sem, main_sem):
    cid = lax.axis_index("c")
    sid = lax.axis_index("s")
    g0 = (sid * 2 + cid) * _CHUNK

    pltpu.async_copy(
        weight_hbm.at[pl.ds(_PAD + 1 + g0, _CHUNK), :], wchunk, setup_sem
    )
    pltpu.async_copy(lengths_hbm, len_v, setup_sem)

    zeros16 = jnp.zeros((16,), jnp.float32)

    def _zero_row(r, carry):
        for kk in range(_EMB_DIM // 16):
            ztile[r, pl.ds(kk * 16, 16)] = zeros16
        return carry

    lax.fori_loop(0, _CHUNK, _zero_row, 0)

    pltpu.make_async_copy(
        weight_hbm.at[pl.ds(0, _CHUNK), :], wchunk, setup_sem
    ).wait()
    pltpu.make_async_copy(lengths_hbm, len_v, setup_sem).wait()

    def _per_batch(b, carry):
        bvec = jnp.broadcast_to(b, (16,)).astype(jnp.int32)
        length = plsc.load_gather(len_v, [bvec])[0]
        cnt = jnp.clip(length - g0, 0, _CHUNK)

        @pl.when(cnt == _CHUNK)
        def _full():
            pltpu.async_copy(
                wchunk, out_hbm.at[b, pl.ds(g0, _CHUNK), :], main_sem
            )

        @pl.when(cnt == 0)
        def _zero():
            pltpu.async_copy(
                ztile, out_hbm.at[b, pl.ds(g0, _CHUNK), :], main_sem
            )

        @pl.when(jnp.logical_and(cnt > 0, cnt < _CHUNK))
        def _boundary():
            off = g0
            rest = _CHUNK - cnt
            for size in _BSIZES:
                take = jnp.bitwise_and(cnt, size)
                cur = off

                @pl.when(take > 0)
                def _copybits(cur=cur, size=size):
                    pltpu.async_copy(
                        wchunk.at[pl.ds(cur - g0, size), :],
                        out_hbm.at[b, pl.ds(cur, size), :],
                        main_sem,
                    )

                off = off + take
            for size in _BSIZES:
                take = jnp.bitwise_and(rest, size)
                cur = off

                @pl.when(take > 0)
                def _zerobits(cur=cur, size=size):
                    pltpu.async_copy(
                        ztile.at[pl.ds(0, size), :],
                        out_hbm.at[b, pl.ds(cur, size), :],
                        main_sem,
                    )

                off = off + take

        return carry

    lax.fori_loop(0, _BSZ, _per_batch, 0)

    pltpu.make_async_copy(
        out_hbm.at[pl.ds(0, _BSZ), pl.ds(0, _CHUNK), :],
        out_hbm.at[pl.ds(0, _BSZ), pl.ds(0, _CHUNK), :],
        main_sem,
    ).wait()


@jax.jit
def _positional_embedding(lengths, weight):
    mesh = plsc.VectorSubcoreMesh(
        core_axis_name="c", subcore_axis_name="s", num_cores=2, num_subcores=16
    )
    return pl.kernel(
        _body,
        out_type=jax.ShapeDtypeStruct((_BSZ, _SEQ, _EMB_DIM), jnp.float32),
        mesh=mesh,
        compiler_params=pltpu.CompilerParams(
            use_tc_tiling_on_sc=False, needs_layout_passes=False
        ),
        scratch_types=[
            pltpu.VMEM((_CHUNK, _EMB_DIM), jnp.float32),
            pltpu.VMEM((_CHUNK, _EMB_DIM), jnp.float32),
            pltpu.VMEM((16,), jnp.int32),
            pltpu.SemaphoreType.DMA,
            pltpu.SemaphoreType.DMA,
        ],
    )(lengths, weight)


def kernel(input, lengths, weight):
    del input
    return _positional_embedding(lengths, weight)
